# baseline (speedup 1.0000x reference)
import functools

import jax
import jax.numpy as jnp
from jax import lax
from jax.experimental import pallas as pl
from jax.experimental.pallas import tpu as pltpu
from jax.experimental.pallas import tpu_sc as plsc

N = 10000
D = 128
E = 320000
EPS = 1e-5

NC = 2
NS = 16
NW = NC * NS
EPW = E // NW
CH = 125
NCHUNK = EPW // CH
IC = 40
NPAD = 10240
RPT = NPAD // NS

_mesh = lambda: plsc.VectorSubcoreMesh(
    core_axis_name="c", subcore_axis_name="s", num_cores=NC, num_subcores=NS)


def _sc_degree(dst3, ones_ch, zeros1):
    @functools.partial(
        pl.kernel,
        out_type=jax.ShapeDtypeStruct((NC, NPAD), jnp.float32),
        mesh=_mesh(),
        scratch_types=[
            pltpu.VMEM((NCHUNK, CH), jnp.int32),
            pltpu.VMEM((CH,), jnp.float32),
            pltpu.VMEM_SHARED((NPAD,), jnp.float32),
        ],
    )
    def k(dst_ref, ones_ref, z_ref, out_ref, dst_v, ones_v, acc_sh):
        c = lax.axis_index("c")
        s = lax.axis_index("s")
        wid = s * NC + c
        pltpu.sync_copy(z_ref.at[pl.ds(s * RPT, RPT)],
                        acc_sh.at[pl.ds(s * RPT, RPT)])
        pltpu.sync_copy(dst_ref.at[wid], dst_v)
        pltpu.sync_copy(ones_ref, ones_v)
        plsc.subcore_barrier()

        def body(j, carry):
            pltpu.sync_copy(ones_v, acc_sh.at[dst_v.at[j]], add=True)
            return carry

        lax.fori_loop(0, NCHUNK, body, 0)
        plsc.subcore_barrier()
        pltpu.sync_copy(acc_sh.at[pl.ds(s * RPT, RPT)],
                        out_ref.at[c, pl.ds(s * RPT, RPT)])

    return k(dst3, ones_ch, zeros1)


def _sc_aggregate(h, src3, dst3, zeros2):
    @functools.partial(
        pl.kernel,
        out_type=jax.ShapeDtypeStruct((NC, NPAD, D), jnp.float32),
        mesh=_mesh(),
        scratch_types=[
            pltpu.VMEM((IC, CH), jnp.int32),
            pltpu.VMEM((IC, CH), jnp.int32),
            pltpu.VMEM((CH, D), jnp.float32),
            pltpu.VMEM((CH, D), jnp.float32),
            pltpu.VMEM_SHARED((NPAD, D), jnp.float32),
            pltpu.SemaphoreType.DMA,
            pltpu.SemaphoreType.DMA,
        ],
    )
    def k(h_ref, src_ref, dst_ref, z_ref, out_ref,
          src_v, dst_v, rows0, rows1, acc_sh, sem0, sem1):
        c = lax.axis_index("c")
        s = lax.axis_index("s")
        wid = s * NC + c
        pltpu.sync_copy(z_ref.at[pl.ds(s * RPT, RPT)],
                        acc_sh.at[pl.ds(s * RPT, RPT)])
        plsc.subcore_barrier()

        def gath(j, buf, sem):
            pltpu.async_copy(h_ref.at[src_v.at[j, pl.ds(0, 64)]],
                             buf.at[pl.ds(0, 64)], sem)
            pltpu.async_copy(h_ref.at[src_v.at[j, pl.ds(64, CH - 64)]],
                             buf.at[pl.ds(64, CH - 64)], sem)

        def gwait(j, buf, sem):
            pltpu.make_async_copy(h_ref.at[src_v.at[j, pl.ds(0, 64)]],
                                  buf.at[pl.ds(0, 64)], sem).wait()
            pltpu.make_async_copy(h_ref.at[src_v.at[j, pl.ds(64, CH - 64)]],
                                  buf.at[pl.ds(64, CH - 64)], sem).wait()

        def grp(g, carry0):
            pltpu.sync_copy(src_ref.at[wid, pl.ds(g * IC, IC)], src_v)
            pltpu.sync_copy(dst_ref.at[wid, pl.ds(g * IC, IC)], dst_v)
            gath(0, rows0, sem0)

            def body(i, carry):
                j0 = 2 * i
                gwait(j0, rows0, sem0)
                gath(j0 + 1, rows1, sem1)
                pltpu.sync_copy(rows0, acc_sh.at[dst_v.at[j0]], add=True)
                gwait(j0 + 1, rows1, sem1)

                @pl.when(j0 + 2 < IC)
                def _():
                    gath(j0 + 2, rows0, sem0)

                pltpu.sync_copy(rows1, acc_sh.at[dst_v.at[j0 + 1]], add=True)
                return carry

            lax.fori_loop(0, IC // 2, body, 0)
            return carry0

        lax.fori_loop(0, NCHUNK // IC, grp, 0)
        plsc.subcore_barrier()
        pltpu.sync_copy(acc_sh.at[pl.ds(s * RPT, RPT)],
                        out_ref.at[c, pl.ds(s * RPT, RPT)])

    return k(h, src3, dst3, zeros2)


_R = 1000
_G = N // _R


def _tc_prep(deg2, features):
    def body(deg_ref, f_ref, dinv_ref, h1_ref):
        deg = jnp.sum(deg_ref[...], axis=1, keepdims=True)
        dinv = lax.rsqrt(jnp.maximum(deg, 1.0))
        dinv_ref[...] = dinv
        h1_ref[...] = f_ref[...] * dinv

    return pl.pallas_call(
        body,
        grid=(_G,),
        in_specs=[
            pl.BlockSpec((_R, 2), lambda i: (i, 0)),
            pl.BlockSpec((_R, D), lambda i: (i, 0)),
        ],
        out_specs=[
            pl.BlockSpec((_R, 1), lambda i: (i, 0)),
            pl.BlockSpec((_R, D), lambda i: (i, 0)),
        ],
        out_shape=[
            jax.ShapeDtypeStruct((N, 1), jnp.float32),
            jax.ShapeDtypeStruct((N, D), jnp.float32),
        ],
    )(deg2, features)


def _tc_conv1bn(features, parts, dinv, W1, b1, gamma, beta):
    def body(f_ref, p_ref, dinv_ref, w_ref, b_ref, g_ref, be_ref,
             xbn_ref, h2_ref, x1_scr, st_scr):
        i = pl.program_id(0)

        @pl.when(i < _G)
        def _():
            agg = p_ref[0] + p_ref[1]
            x1 = -(agg * dinv_ref[...])
            t = jnp.dot(f_ref[...], w_ref[0:D, :],
                        preferred_element_type=jnp.float32)
            t += jnp.dot(x1, w_ref[D:2 * D, :],
                         preferred_element_type=jnp.float32)
            x = jnp.maximum(t + b_ref[...], 0.0)
            x1_scr[pl.ds(i * _R, _R), :] = x
            loc = jnp.concatenate(
                [jnp.sum(x, axis=0, keepdims=True),
                 jnp.sum(x * x, axis=0, keepdims=True)], axis=0)

            @pl.when(i == 0)
            def _():
                st_scr[...] = loc

            @pl.when(i > 0)
            def _():
                st_scr[...] += loc

        @pl.when(i >= _G)
        def _():
            mu = st_scr[0:1, :] * (1.0 / N)
            var = st_scr[1:2, :] * (1.0 / N) - mu * mu
            a = lax.rsqrt(var + EPS) * g_ref[...]
            bcol = be_ref[...] - a * mu
            x = x1_scr[pl.ds((i - _G) * _R, _R), :]
            xbn = a * x + bcol
            xbn_ref[...] = xbn
            h2_ref[...] = xbn * dinv_ref[...]

    ph = lambda i: jnp.where(i < _G, i, 0)
    return pl.pallas_call(
        body,
        grid=(2 * _G,),
        in_specs=[
            pl.BlockSpec((_R, D), lambda i: (ph(i), 0)),
            pl.BlockSpec((NC, _R, D), lambda i: (0, ph(i), 0)),
            pl.BlockSpec((_R, 1), lambda i: (lax.rem(i, _G), 0)),
            pl.BlockSpec((2 * D, D), lambda i: (0, 0)),
            pl.BlockSpec((1, D), lambda i: (0, 0)),
            pl.BlockSpec((1, D), lambda i: (0, 0)),
            pl.BlockSpec((1, D), lambda i: (0, 0)),
        ],
        out_specs=[
            pl.BlockSpec((_R, D), lambda i: (jnp.maximum(i - _G, 0), 0)),
            pl.BlockSpec((_R, D), lambda i: (jnp.maximum(i - _G, 0), 0)),
        ],
        out_shape=[
            jax.ShapeDtypeStruct((N, D), jnp.float32),
            jax.ShapeDtypeStruct((N, D), jnp.float32),
        ],
        scratch_shapes=[
            pltpu.VMEM((N, D), jnp.float32),
            pltpu.VMEM((2, D), jnp.float32),
        ],
    )(features, parts, dinv, W1, b1, gamma, beta)


def _tc_final(xbn, parts, dinv, W2, b2, Wm1, bm1, Wm2, bm2):
    def body(x_ref, p_ref, dinv_ref, w2_ref, b2_ref,
             wm1_ref, bm1_ref, wm2_ref, bm2_ref, out_ref):
        xbn = x_ref[...]
        agg = p_ref[0] + p_ref[1]
        x1 = -(agg * dinv_ref[...])
        t = jnp.dot(xbn, w2_ref[0:D, :], preferred_element_type=jnp.float32)
        t += jnp.dot(x1, w2_ref[D:2 * D, :], preferred_element_type=jnp.float32)
        x = jnp.maximum(t + b2_ref[...], 0.0) + xbn
        h = jnp.maximum(
            jnp.dot(x, wm1_ref[...], preferred_element_type=jnp.float32)
            + bm1_ref[...], 0.0)
        out_ref[...] = (jnp.dot(h, wm2_ref[...],
                                preferred_element_type=jnp.float32)
                        + bm2_ref[...])

    return pl.pallas_call(
        body,
        grid=(_G,),
        in_specs=[
            pl.BlockSpec((_R, D), lambda i: (i, 0)),
            pl.BlockSpec((NC, _R, D), lambda i: (0, i, 0)),
            pl.BlockSpec((_R, 1), lambda i: (i, 0)),
            pl.BlockSpec((2 * D, D), lambda i: (0, 0)),
            pl.BlockSpec((1, D), lambda i: (0, 0)),
            pl.BlockSpec((D, D), lambda i: (0, 0)),
            pl.BlockSpec((1, D), lambda i: (0, 0)),
            pl.BlockSpec((D, D), lambda i: (0, 0)),
            pl.BlockSpec((1, D), lambda i: (0, 0)),
        ],
        out_specs=pl.BlockSpec((_R, D), lambda i: (i, 0)),
        out_shape=jax.ShapeDtypeStruct((N, D), jnp.float32),
    )(xbn, parts, dinv, W2, b2, Wm1, bm1, Wm2, bm2)


def kernel(features, edge_index, W1, b1, W2, b2, gamma, beta, Wm1, bm1, Wm2, bm2):
    src3 = edge_index[0].reshape(NW, NCHUNK, CH)
    dst3 = edge_index[1].reshape(NW, NCHUNK, CH)
    ones_ch = jnp.ones((CH,), jnp.float32)
    zeros1 = jnp.zeros((NPAD,), jnp.float32)
    zeros2 = jnp.zeros((NPAD, D), jnp.float32)

    deg_parts = _sc_degree(dst3, ones_ch, zeros1)
    deg2 = deg_parts[:, :N].T

    dinv, h1 = _tc_prep(deg2, features)
    parts1 = _sc_aggregate(h1, src3, dst3, zeros2)
    xbn, h2 = _tc_conv1bn(features, parts1, dinv, W1, b1.reshape(1, D),
                          gamma.reshape(1, D), beta.reshape(1, D))
    parts2 = _sc_aggregate(h2, src3, dst3, zeros2)
    return _tc_final(xbn, parts2, dinv, W2, b2.reshape(1, D),
                     Wm1, bm1.reshape(1, D), Wm2, bm2.reshape(1, D))


# speedup vs baseline: 1.0069x; 1.0069x over previous
import functools

import jax
import jax.numpy as jnp
from jax import lax
from jax.experimental import pallas as pl
from jax.experimental.pallas import tpu as pltpu
from jax.experimental.pallas import tpu_sc as plsc

N = 10000
D = 128
E = 320000
EPS = 1e-5

NC = 2
NS = 16
NW = NC * NS
EPW = E // NW
CH = 125
NCHUNK = EPW // CH
IC = 40
NPAD = 10240
RPT = NPAD // NS

_mesh = lambda: plsc.VectorSubcoreMesh(
    core_axis_name="c", subcore_axis_name="s", num_cores=NC, num_subcores=NS)


def _sc_degree(dst3, ones_ch, zeros1):
    @functools.partial(
        pl.kernel,
        out_type=jax.ShapeDtypeStruct((NC, NPAD), jnp.float32),
        mesh=_mesh(),
        scratch_types=[
            pltpu.VMEM((NCHUNK, CH), jnp.int32),
            pltpu.VMEM((CH,), jnp.float32),
            pltpu.VMEM_SHARED((NPAD,), jnp.float32),
        ],
    )
    def k(dst_ref, ones_ref, z_ref, out_ref, dst_v, ones_v, acc_sh):
        c = lax.axis_index("c")
        s = lax.axis_index("s")
        wid = s * NC + c
        pltpu.sync_copy(z_ref.at[pl.ds(s * RPT, RPT)],
                        acc_sh.at[pl.ds(s * RPT, RPT)])
        pltpu.sync_copy(dst_ref.at[wid], dst_v)
        pltpu.sync_copy(ones_ref, ones_v)
        plsc.subcore_barrier()

        def body(j, carry):
            pltpu.sync_copy(ones_v, acc_sh.at[dst_v.at[j]], add=True)
            return carry

        lax.fori_loop(0, NCHUNK, body, 0)
        plsc.subcore_barrier()
        pltpu.sync_copy(acc_sh.at[pl.ds(s * RPT, RPT)],
                        out_ref.at[c, pl.ds(s * RPT, RPT)])

    return k(dst3, ones_ch, zeros1)


def _sc_aggregate(h, src3, dst3, zeros2):
    @functools.partial(
        pl.kernel,
        out_type=jax.ShapeDtypeStruct((NC, NPAD, D), jnp.float32),
        mesh=_mesh(),
        scratch_types=[
            pltpu.VMEM((IC, CH), jnp.int32),
            pltpu.VMEM((IC, CH), jnp.int32),
            pltpu.VMEM((CH, D), jnp.float32),
            pltpu.VMEM((CH, D), jnp.float32),
            pltpu.VMEM_SHARED((NPAD, D), jnp.float32),
            pltpu.SemaphoreType.DMA,
            pltpu.SemaphoreType.DMA,
        ],
    )
    def k(h_ref, src_ref, dst_ref, z_ref, out_ref,
          src_v, dst_v, rows0, rows1, acc_sh, sem0, sem1):
        c = lax.axis_index("c")
        s = lax.axis_index("s")
        wid = s * NC + c
        pltpu.sync_copy(z_ref.at[pl.ds(s * RPT, RPT)],
                        acc_sh.at[pl.ds(s * RPT, RPT)])
        plsc.subcore_barrier()

        def grp(g, carry0):
            pltpu.sync_copy(src_ref.at[wid, pl.ds(g * IC, IC)], src_v)
            pltpu.sync_copy(dst_ref.at[wid, pl.ds(g * IC, IC)], dst_v)
            pltpu.async_copy(h_ref.at[src_v.at[0]], rows0, sem0)

            def body(i, carry):
                j0 = 2 * i
                pltpu.make_async_copy(
                    h_ref.at[src_v.at[j0]], rows0, sem0).wait()
                pltpu.async_copy(h_ref.at[src_v.at[j0 + 1]], rows1, sem1)
                pltpu.sync_copy(rows0, acc_sh.at[dst_v.at[j0]], add=True)
                pltpu.make_async_copy(
                    h_ref.at[src_v.at[j0 + 1]], rows1, sem1).wait()

                @pl.when(j0 + 2 < IC)
                def _():
                    pltpu.async_copy(h_ref.at[src_v.at[j0 + 2]], rows0, sem0)

                pltpu.sync_copy(rows1, acc_sh.at[dst_v.at[j0 + 1]], add=True)
                return carry

            lax.fori_loop(0, IC // 2, body, 0)
            return carry0

        lax.fori_loop(0, NCHUNK // IC, grp, 0)
        plsc.subcore_barrier()
        pltpu.sync_copy(acc_sh.at[pl.ds(s * RPT, RPT)],
                        out_ref.at[c, pl.ds(s * RPT, RPT)])

    return k(h, src3, dst3, zeros2)


_R = 1000
_G = N // _R


def _tc_prep(deg2, features):
    def body(deg_ref, f_ref, dinv_ref, h1_ref):
        deg = jnp.sum(deg_ref[...], axis=1, keepdims=True)
        dinv = lax.rsqrt(jnp.maximum(deg, 1.0))
        dinv_ref[...] = dinv
        h1_ref[...] = f_ref[...] * dinv

    return pl.pallas_call(
        body,
        grid=(_G,),
        in_specs=[
            pl.BlockSpec((_R, 2), lambda i: (i, 0)),
            pl.BlockSpec((_R, D), lambda i: (i, 0)),
        ],
        out_specs=[
            pl.BlockSpec((_R, 1), lambda i: (i, 0)),
            pl.BlockSpec((_R, D), lambda i: (i, 0)),
        ],
        out_shape=[
            jax.ShapeDtypeStruct((N, 1), jnp.float32),
            jax.ShapeDtypeStruct((N, D), jnp.float32),
        ],
    )(deg2, features)


def _tc_conv1bn(features, parts, dinv, W1, b1, gamma, beta):
    def body(f_ref, p_ref, dinv_ref, w_ref, b_ref, g_ref, be_ref,
             xbn_ref, h2_ref, x1_scr, st_scr):
        i = pl.program_id(0)

        @pl.when(i < _G)
        def _():
            agg = p_ref[0] + p_ref[1]
            x1 = -(agg * dinv_ref[...])
            t = jnp.dot(f_ref[...], w_ref[0:D, :],
                        preferred_element_type=jnp.float32)
            t += jnp.dot(x1, w_ref[D:2 * D, :],
                         preferred_element_type=jnp.float32)
            x = jnp.maximum(t + b_ref[...], 0.0)
            x1_scr[pl.ds(i * _R, _R), :] = x
            loc = jnp.concatenate(
                [jnp.sum(x, axis=0, keepdims=True),
                 jnp.sum(x * x, axis=0, keepdims=True)], axis=0)

            @pl.when(i == 0)
            def _():
                st_scr[...] = loc

            @pl.when(i > 0)
            def _():
                st_scr[...] += loc

        @pl.when(i >= _G)
        def _():
            mu = st_scr[0:1, :] * (1.0 / N)
            var = st_scr[1:2, :] * (1.0 / N) - mu * mu
            a = lax.rsqrt(var + EPS) * g_ref[...]
            bcol = be_ref[...] - a * mu
            x = x1_scr[pl.ds((i - _G) * _R, _R), :]
            xbn = a * x + bcol
            xbn_ref[...] = xbn
            h2_ref[...] = xbn * dinv_ref[...]

    ph = lambda i: jnp.where(i < _G, i, 0)
    return pl.pallas_call(
        body,
        grid=(2 * _G,),
        in_specs=[
            pl.BlockSpec((_R, D), lambda i: (ph(i), 0)),
            pl.BlockSpec((NC, _R, D), lambda i: (0, ph(i), 0)),
            pl.BlockSpec((_R, 1), lambda i: (lax.rem(i, _G), 0)),
            pl.BlockSpec((2 * D, D), lambda i: (0, 0)),
            pl.BlockSpec((1, D), lambda i: (0, 0)),
            pl.BlockSpec((1, D), lambda i: (0, 0)),
            pl.BlockSpec((1, D), lambda i: (0, 0)),
        ],
        out_specs=[
            pl.BlockSpec((_R, D), lambda i: (jnp.maximum(i - _G, 0), 0)),
            pl.BlockSpec((_R, D), lambda i: (jnp.maximum(i - _G, 0), 0)),
        ],
        out_shape=[
            jax.ShapeDtypeStruct((N, D), jnp.float32),
            jax.ShapeDtypeStruct((N, D), jnp.float32),
        ],
        scratch_shapes=[
            pltpu.VMEM((N, D), jnp.float32),
            pltpu.VMEM((2, D), jnp.float32),
        ],
    )(features, parts, dinv, W1, b1, gamma, beta)


def _tc_final(xbn, parts, dinv, W2, b2, Wm1, bm1, Wm2, bm2):
    def body(x_ref, p_ref, dinv_ref, w2_ref, b2_ref,
             wm1_ref, bm1_ref, wm2_ref, bm2_ref, out_ref):
        xbn = x_ref[...]
        agg = p_ref[0] + p_ref[1]
        x1 = -(agg * dinv_ref[...])
        t = jnp.dot(xbn, w2_ref[0:D, :], preferred_element_type=jnp.float32)
        t += jnp.dot(x1, w2_ref[D:2 * D, :], preferred_element_type=jnp.float32)
        x = jnp.maximum(t + b2_ref[...], 0.0) + xbn
        h = jnp.maximum(
            jnp.dot(x, wm1_ref[...], preferred_element_type=jnp.float32)
            + bm1_ref[...], 0.0)
        out_ref[...] = (jnp.dot(h, wm2_ref[...],
                                preferred_element_type=jnp.float32)
                        + bm2_ref[...])

    return pl.pallas_call(
        body,
        grid=(_G,),
        in_specs=[
            pl.BlockSpec((_R, D), lambda i: (i, 0)),
            pl.BlockSpec((NC, _R, D), lambda i: (0, i, 0)),
            pl.BlockSpec((_R, 1), lambda i: (i, 0)),
            pl.BlockSpec((2 * D, D), lambda i: (0, 0)),
            pl.BlockSpec((1, D), lambda i: (0, 0)),
            pl.BlockSpec((D, D), lambda i: (0, 0)),
            pl.BlockSpec((1, D), lambda i: (0, 0)),
            pl.BlockSpec((D, D), lambda i: (0, 0)),
            pl.BlockSpec((1, D), lambda i: (0, 0)),
        ],
        out_specs=pl.BlockSpec((_R, D), lambda i: (i, 0)),
        out_shape=jax.ShapeDtypeStruct((N, D), jnp.float32),
    )(xbn, parts, dinv, W2, b2, Wm1, bm1, Wm2, bm2)


def kernel(features, edge_index, W1, b1, W2, b2, gamma, beta, Wm1, bm1, Wm2, bm2):
    src3 = edge_index[0].reshape(NW, NCHUNK, CH)
    dst3 = edge_index[1].reshape(NW, NCHUNK, CH)
    ones_ch = jnp.ones((CH,), jnp.float32)
    zeros1 = jnp.zeros((NPAD,), jnp.float32)
    zeros2 = jnp.zeros((NPAD, D), jnp.float32)

    deg_parts = _sc_degree(dst3, ones_ch, zeros1)
    deg2 = deg_parts[:, :N].T

    dinv, h1 = _tc_prep(deg2, features)
    parts1 = _sc_aggregate(h1, src3, dst3, zeros2)
    xbn, h2 = _tc_conv1bn(features, parts1, dinv, W1, b1.reshape(1, D),
                          gamma.reshape(1, D), beta.reshape(1, D))
    parts2 = _sc_aggregate(h2, src3, dst3, zeros2)
    return _tc_final(xbn, parts2, dinv, W2, b2.reshape(1, D),
                     Wm1, bm1.reshape(1, D), Wm2, bm2.reshape(1, D))
